# baseline (speedup 1.0000x reference)
import functools

import jax
import jax.numpy as jnp
from jax import lax
from jax.experimental import pallas as pl
from jax.experimental.pallas import tpu as pltpu
from jax.experimental.pallas import tpu_sc as plsc

NC = 2
NS = 16
L = 16
NW = NC * NS
RPW = 32
KP = 16
MW = 3136


def _sc_extract(ids_t, B, S, opbits, tdf_pad, iota16):
    mesh = plsc.VectorSubcoreMesh(core_axis_name="c", subcore_axis_name="s")

    @functools.partial(
        pl.kernel,
        out_type=[
            jax.ShapeDtypeStruct((B, KP), jnp.float32),
            jax.ShapeDtypeStruct((B, KP), jnp.float32),
        ],
        mesh=mesh,
        compiler_params=pltpu.CompilerParams(
            needs_layout_passes=False, use_tc_tiling_on_sc=False),
        scratch_types=[
            pltpu.VMEM((RPW * S,), jnp.int32),
            pltpu.VMEM((MW,), jnp.int32),
            pltpu.VMEM((L,), jnp.int32),
            pltpu.VMEM((RPW, KP), jnp.float32),
            pltpu.VMEM((RPW, KP), jnp.float32),
            pltpu.VMEM((RPW,), jnp.int32),
            pltpu.SemaphoreType.DMA,
        ],
    )
    def sc_kernel(ids_hbm, opbits_hbm, tdf_hbm, iota_hbm, da_hbm, db_hbm,
                  ids_v, mask_v, iota_v, da_v, db_v, st_min, sem):
        wid = lax.axis_index("s") * NC + lax.axis_index("c")
        base = wid * RPW
        pltpu.sync_copy(ids_hbm.at[pl.ds(wid * S * RPW, S * RPW)], ids_v)
        pltpu.sync_copy(opbits_hbm, mask_v)
        pltpu.sync_copy(iota_hbm, iota_v)
        for g in range(RPW // L):
            off = g * L
            sl = pl.ds(off, L)
            st_min[sl] = jnp.full((L,), S, jnp.int32)

            def body(s, c):
                cur = ids_v[pl.ds(s * RPW + off, L)]
                word = plsc.load_gather(
                    mask_v, [lax.shift_right_logical(cur, 5)])
                bit = lax.shift_right_logical(word, cur & 31) & 1
                st_min[sl] = jnp.minimum(
                    st_min[sl], jnp.where(bit != 0, s, S))
                return c

            lax.fori_loop(0, S, body, 0)
            minpos = st_min[sl]
            pos = jnp.where(minpos >= S, 0, minpos)
            lane = iota_v[...] + off
            a_ids = plsc.load_gather(
                ids_v, [jnp.maximum(pos - 1, 0) * RPW + lane])
            b_ids = plsc.load_gather(
                ids_v, [jnp.minimum(pos + 1, S - 1) * RPW + lane])
            pltpu.async_copy(tdf_hbm.at[a_ids], da_v.at[sl], sem).wait()
            pltpu.async_copy(tdf_hbm.at[b_ids], db_v.at[sl], sem).wait()
        pltpu.sync_copy(da_v, da_hbm.at[pl.ds(base, RPW)])
        pltpu.sync_copy(db_v, db_hbm.at[pl.ds(base, RPW)])

    return sc_kernel(ids_t, opbits, tdf_pad, iota16)


def _tc_broadcast(da_flat, db_flat, S, K):
    B = da_flat.shape[0]
    BB = 16

    def body(da_ref, db_ref, oa_ref, ob_ref):
        for r in range(BB):
            oa_ref[r] = jnp.broadcast_to(da_ref[r][None, :K], (S, K))
            ob_ref[r] = jnp.broadcast_to(db_ref[r][None, :K], (S, K))

    return pl.pallas_call(
        body,
        grid=(B // BB,),
        in_specs=[
            pl.BlockSpec((BB, KP), lambda i: (i, 0)),
            pl.BlockSpec((BB, KP), lambda i: (i, 0)),
        ],
        out_specs=[
            pl.BlockSpec((BB, S, K), lambda i: (i, 0, 0)),
            pl.BlockSpec((BB, S, K), lambda i: (i, 0, 0)),
        ],
        out_shape=[
            jax.ShapeDtypeStruct((B, S, K), jnp.float32),
            jax.ShapeDtypeStruct((B, S, K), jnp.float32),
        ],
    )(da_flat, db_flat)


def kernel(h, input_ids, attention_mask, is_operator, token_digit_value,
           token_digits_full):
    B, S = input_ids.shape
    Vv, K = token_digits_full.shape
    ids_t = input_ids.reshape(NW, RPW, S).transpose(0, 2, 1).reshape(-1)
    ops = is_operator.astype(jnp.uint32).reshape(-1, 32)
    words = (ops << jnp.arange(32, dtype=jnp.uint32)[None, :]).sum(
        axis=1, dtype=jnp.uint32)
    opbits = lax.bitcast_convert_type(
        jnp.zeros((MW,), jnp.uint32).at[:words.shape[0]].set(words),
        jnp.int32)
    tdf_pad = jnp.pad(token_digits_full, ((0, 0), (0, KP - K)))
    iota16 = jnp.arange(L, dtype=jnp.int32)
    da_flat, db_flat = _sc_extract(ids_t, B, S, opbits, tdf_pad, iota16)
    d_a, d_b = _tc_broadcast(da_flat, db_flat, S, K)
    return (d_a, d_b, d_a, d_b)


# speedup vs baseline: 1.2182x; 1.2182x over previous
import functools

import jax
import jax.numpy as jnp
from jax import lax
from jax.experimental import pallas as pl
from jax.experimental.pallas import tpu as pltpu
from jax.experimental.pallas import tpu_sc as plsc

NC = 2
NS = 16
L = 16
NW = NC * NS
RPW = 32
KP = 16
MW = 3136
NPAT = 5


def _sc_extract_broadcast(ids_t, B, S, K, opbits, tdf_pad, pidx):
    SK = S * K
    mesh = plsc.VectorSubcoreMesh(core_axis_name="c", subcore_axis_name="s")

    @functools.partial(
        pl.kernel,
        out_type=[
            jax.ShapeDtypeStruct((B, SK), jnp.float32),
            jax.ShapeDtypeStruct((B, SK), jnp.float32),
        ],
        mesh=mesh,
        compiler_params=pltpu.CompilerParams(
            needs_layout_passes=False, use_tc_tiling_on_sc=False),
        scratch_types=[
            pltpu.VMEM((RPW * S,), jnp.int32),
            pltpu.VMEM((MW,), jnp.int32),
            pltpu.VMEM(((NPAT + 1) * L,), jnp.int32),
            pltpu.VMEM((RPW, KP), jnp.float32),
            pltpu.VMEM((RPW, KP), jnp.float32),
            pltpu.VMEM((RPW,), jnp.int32),
            pltpu.VMEM((RPW, SK), jnp.float32),
            pltpu.SemaphoreType.DMA,
        ],
    )
    def sc_kernel(ids_hbm, opbits_hbm, tdf_hbm, pidx_hbm, oa_hbm, ob_hbm,
                  ids_v, mask_v, pidx_v, da_v, db_v, st_min, buf, sem):
        wid = lax.axis_index("s") * NC + lax.axis_index("c")
        base = wid * RPW
        pltpu.sync_copy(ids_hbm.at[pl.ds(wid * S * RPW, S * RPW)], ids_v)
        pltpu.sync_copy(opbits_hbm, mask_v)
        pltpu.sync_copy(pidx_hbm, pidx_v)
        for g in range(RPW // L):
            off = g * L
            sl = pl.ds(off, L)
            st_min[sl] = jnp.full((L,), S, jnp.int32)

            def body(s, c):
                cur = ids_v[pl.ds(s * RPW + off, L)]
                word = plsc.load_gather(
                    mask_v, [lax.shift_right_logical(cur, 5)])
                bit = lax.shift_right_logical(word, cur & 31) & 1
                st_min[sl] = jnp.minimum(
                    st_min[sl], jnp.where(bit != 0, s, S))
                return c

            lax.fori_loop(0, S, body, 0)
            minpos = st_min[sl]
            pos = jnp.where(minpos >= S, 0, minpos)
            lane = pidx_v[pl.ds(NPAT * L, L)] + off
            a_ids = plsc.load_gather(
                ids_v, [jnp.maximum(pos - 1, 0) * RPW + lane])
            b_ids = plsc.load_gather(
                ids_v, [jnp.minimum(pos + 1, S - 1) * RPW + lane])
            pltpu.async_copy(tdf_hbm.at[a_ids], da_v.at[sl], sem).wait()
            pltpu.async_copy(tdf_hbm.at[b_ids], db_v.at[sl], sem).wait()

        def build(src_v, out_hbm):
            def row(rr, c):
                rvec = jnp.zeros((L,), jnp.int32) + rr
                for j in range(NPAT):
                    pat = plsc.load_gather(
                        src_v, [rvec, pidx_v[pl.ds(j * L, L)]])
                    for c2 in range(j, SK // L, NPAT):
                        buf[rr, pl.ds(c2 * L, L)] = pat
                return c

            lax.fori_loop(0, RPW, row, 0)
            pltpu.sync_copy(buf, out_hbm.at[pl.ds(base, RPW)])

        build(da_v, oa_hbm)
        build(db_v, ob_hbm)

    return sc_kernel(ids_t, opbits, tdf_pad, pidx)


def kernel(h, input_ids, attention_mask, is_operator, token_digit_value,
           token_digits_full):
    B, S = input_ids.shape
    Vv, K = token_digits_full.shape
    ids_t = input_ids.reshape(NW, RPW, S).transpose(0, 2, 1).reshape(-1)
    ops = is_operator.astype(jnp.uint32).reshape(-1, 32)
    words = (ops << jnp.arange(32, dtype=jnp.uint32)[None, :]).sum(
        axis=1, dtype=jnp.uint32)
    opbits = lax.bitcast_convert_type(
        jnp.zeros((MW,), jnp.uint32).at[:words.shape[0]].set(words),
        jnp.int32)
    tdf_pad = jnp.pad(token_digits_full, ((0, 0), (0, KP - K)))
    pidx = jnp.concatenate([
        jnp.arange(NPAT * L, dtype=jnp.int32) % K,
        jnp.arange(L, dtype=jnp.int32)])
    oa, ob = _sc_extract_broadcast(ids_t, B, S, K, opbits, tdf_pad, pidx)
    d_a = oa.reshape(B, S, K)
    d_b = ob.reshape(B, S, K)
    return (d_a, d_b, d_a, d_b)


# speedup vs baseline: 2.2619x; 1.8567x over previous
import functools

import jax
import jax.numpy as jnp
from jax import lax
from jax.experimental import pallas as pl
from jax.experimental.pallas import tpu as pltpu
from jax.experimental.pallas import tpu_sc as plsc

NC = 2
NS = 16
L = 16
NW = NC * NS
RPW = 32
KP = 16
MW = 3136


def _sc_extract(ids_t, B, S, opbits, tdf_pad, iota16):
    mesh = plsc.VectorSubcoreMesh(core_axis_name="c", subcore_axis_name="s")

    @functools.partial(
        pl.kernel,
        out_type=[
            jax.ShapeDtypeStruct((B, KP), jnp.float32),
            jax.ShapeDtypeStruct((B, KP), jnp.float32),
        ],
        mesh=mesh,
        compiler_params=pltpu.CompilerParams(
            needs_layout_passes=False, use_tc_tiling_on_sc=False),
        scratch_types=[
            pltpu.VMEM((RPW * S,), jnp.int32),
            pltpu.VMEM((MW,), jnp.int32),
            pltpu.VMEM((L,), jnp.int32),
            pltpu.VMEM((RPW, KP), jnp.float32),
            pltpu.VMEM((RPW, KP), jnp.float32),
            pltpu.VMEM((RPW,), jnp.int32),
            pltpu.SemaphoreType.DMA,
        ],
    )
    def sc_kernel(ids_hbm, opbits_hbm, tdf_hbm, iota_hbm, da_hbm, db_hbm,
                  ids_v, mask_v, iota_v, da_v, db_v, st_min, sem):
        wid = lax.axis_index("s") * NC + lax.axis_index("c")
        base = wid * RPW
        pltpu.sync_copy(ids_hbm.at[pl.ds(wid * S * RPW, S * RPW)], ids_v)
        pltpu.sync_copy(opbits_hbm, mask_v)
        pltpu.sync_copy(iota_hbm, iota_v)
        for g in range(RPW // L):
            off = g * L
            sl = pl.ds(off, L)
            st_min[sl] = jnp.full((L,), S, jnp.int32)

            def body(s, c):
                cur = ids_v[pl.ds(s * RPW + off, L)]
                word = plsc.load_gather(
                    mask_v, [lax.shift_right_logical(cur, 5)])
                bit = lax.shift_right_logical(word, cur & 31) & 1
                st_min[sl] = jnp.minimum(
                    st_min[sl], jnp.where(bit != 0, s, S))
                return c

            lax.fori_loop(0, S, body, 0)
            minpos = st_min[sl]
            pos = jnp.where(minpos >= S, 0, minpos)
            lane = iota_v[...] + off
            a_ids = plsc.load_gather(
                ids_v, [jnp.maximum(pos - 1, 0) * RPW + lane])
            b_ids = plsc.load_gather(
                ids_v, [jnp.minimum(pos + 1, S - 1) * RPW + lane])
            pltpu.async_copy(tdf_hbm.at[a_ids], da_v.at[sl], sem).wait()
            pltpu.async_copy(tdf_hbm.at[b_ids], db_v.at[sl], sem).wait()
        pltpu.sync_copy(da_v, da_hbm.at[pl.ds(base, RPW)])
        pltpu.sync_copy(db_v, db_hbm.at[pl.ds(base, RPW)])

    return sc_kernel(ids_t, opbits, tdf_pad, iota16)


def kernel(h, input_ids, attention_mask, is_operator, token_digit_value,
           token_digits_full):
    B, S = input_ids.shape
    Vv, K = token_digits_full.shape
    ids_t = input_ids.reshape(NW, RPW, S).transpose(0, 2, 1).reshape(-1)
    ops = is_operator.astype(jnp.uint32).reshape(-1, 32)
    words = (ops << jnp.arange(32, dtype=jnp.uint32)[None, :]).sum(
        axis=1, dtype=jnp.uint32)
    opbits = lax.bitcast_convert_type(
        jnp.zeros((MW,), jnp.uint32).at[:words.shape[0]].set(words),
        jnp.int32)
    tdf_pad = jnp.pad(token_digits_full, ((0, 0), (0, KP - K)))
    iota16 = jnp.arange(L, dtype=jnp.int32)
    da_flat, db_flat = _sc_extract(ids_t, B, S, opbits, tdf_pad, iota16)
    d_a = jnp.broadcast_to(da_flat[:, None, :K], (B, S, K))
    d_b = jnp.broadcast_to(db_flat[:, None, :K], (B, S, K))
    return (d_a, d_b, d_a, d_b)
